# baseline (speedup 1.0000x reference)
import functools

import jax
import jax.numpy as jnp
from jax import lax
from jax.experimental import pallas as pl
from jax.experimental.pallas import tpu as pltpu
from jax.experimental.pallas import tpu_sc as plsc

_NC = 2
_NS = 16
_NW = _NC * _NS


def _make_gather(n_rows, h, w, b, c_dim):
    b_per_w = b // _NW
    mesh = plsc.VectorSubcoreMesh(
        core_axis_name="c", subcore_axis_name="s", num_cores=_NC,
        num_subcores=_NS)

    @functools.partial(
        pl.kernel,
        mesh=mesh,
        out_type=[
            jax.ShapeDtypeStruct((b, h, w), jnp.float32),
            jax.ShapeDtypeStruct((b, c_dim), jnp.float32),
        ],
        scratch_types=[
            pltpu.SMEM((b_per_w,), jnp.int32),
            pltpu.VMEM((b_per_w,), jnp.int32),
            pltpu.VMEM_SHARED((_NS, b_per_w), jnp.int32),
            pltpu.VMEM((b_per_w, c_dim), jnp.float32),
            pltpu.SemaphoreType.DMA,
            pltpu.SemaphoreType.DMA,
        ],
    )
    def gather_kernel(spec_hbm, coords_hbm, idx_hbm, out_hbm, lab_hbm,
                      idx_s, idx_v, idx_sh, crows_v, sem, csem):
        wid = lax.axis_index("s") * _NC + lax.axis_index("c")
        sid = lax.axis_index("s")
        base = wid * b_per_w

        pltpu.sync_copy(idx_hbm.at[pl.ds(base, b_per_w)], idx_v)
        pltpu.sync_copy(idx_v, idx_sh.at[sid])
        pltpu.sync_copy(idx_sh.at[sid], idx_s)
        ccopy = pltpu.async_copy(coords_hbm.at[idx_v], crows_v, csem)

        def body(i, _):
            row = idx_s[i]
            pltpu.async_copy(
                spec_hbm.at[pl.ds(row, 1)],
                out_hbm.at[pl.ds(base + i, 1)], sem)
            return ()

        lax.fori_loop(0, b_per_w, body, ())
        pltpu.make_async_copy(
            spec_hbm.at[pl.ds(0, b_per_w)],
            out_hbm.at[pl.ds(base, b_per_w)], sem).wait()

        ccopy.wait()
        pltpu.sync_copy(crows_v, lab_hbm.at[pl.ds(base, b_per_w)])

    return gather_kernel


def kernel(spectrograms, coords, indices):
    n, h, w = spectrograms.shape
    b = indices.shape[0]
    c_dim = coords.shape[1]
    c_pad = 128
    coords_p = jnp.pad(coords, ((0, 0), (0, c_pad - c_dim)))
    samples, labels = _make_gather(n, h, w, b, c_pad)(
        spectrograms, coords_p, indices)
    return samples[:, None, :, :], labels[:, :c_dim]
